# baseline (speedup 1.0000x reference)
import functools

import jax
import jax.numpy as jnp
from jax import lax
from jax.experimental import pallas as pl
from jax.experimental.pallas import tpu as pltpu
from jax.experimental.pallas import tpu_sc as plsc

_NUM_WORKERS = 32


def kernel(inputs, token_table, pos_table):
    B, L = inputs.shape
    V, E = token_table.shape
    rows_per_w = B // _NUM_WORKERS

    c0 = min(128, L)
    c1 = L - c0

    mesh = plsc.VectorSubcoreMesh(core_axis_name="c", subcore_axis_name="s")

    @functools.partial(
        pl.kernel,
        mesh=mesh,
        compiler_params=pltpu.CompilerParams(use_tc_tiling_on_sc=False),
        out_type=jax.ShapeDtypeStruct((B, L, E), jnp.float32),
        scratch_types=[
            pltpu.VMEM((L,), jnp.int32),
            pltpu.VMEM_SHARED((L, E), jnp.float32),
            pltpu.VMEM((L, E), jnp.float32),
            pltpu.SemaphoreType.DMA,
        ],
    )
    def emb_kernel(inputs_hbm, table_hbm, pos_hbm, out_hbm,
                   idx_v, pos_sh, buf_v, sem):
        sid = lax.axis_index("s")
        wid = sid * 2 + lax.axis_index("c")
        base = wid * rows_per_w

        @pl.when(sid == 0)
        def _stage_pos():
            pltpu.sync_copy(pos_hbm, pos_sh)

        plsc.subcore_barrier()

        def body(i, carry):
            b = base + i
            pltpu.sync_copy(inputs_hbm.at[b], idx_v)
            pltpu.sync_copy(pos_sh, buf_v)
            cp0 = pltpu.async_copy(
                table_hbm.at[idx_v.at[pl.ds(0, c0)]],
                buf_v.at[pl.ds(0, c0)], sem, add=True)
            if c1 > 0:
                cp1 = pltpu.async_copy(
                    table_hbm.at[idx_v.at[pl.ds(c0, c1)]],
                    buf_v.at[pl.ds(c0, c1)], sem, add=True)
            cp0.wait()
            if c1 > 0:
                cp1.wait()
            pltpu.sync_copy(buf_v, out_hbm.at[b])
            return carry

        lax.fori_loop(0, rows_per_w, body, 0)

    return emb_kernel(inputs, token_table, pos_table)


# speedup vs baseline: 1.1748x; 1.1748x over previous
import functools

import jax
import jax.numpy as jnp
from jax import lax
from jax.experimental import pallas as pl
from jax.experimental.pallas import tpu as pltpu
from jax.experimental.pallas import tpu_sc as plsc

_NUM_WORKERS = 32
_NBUF = 4


def kernel(inputs, token_table, pos_table):
    B, L = inputs.shape
    V, E = token_table.shape
    rows_per_w = B // _NUM_WORKERS
    blocks = rows_per_w // _NBUF

    c0 = min(128, L)
    c1 = L - c0

    mesh = plsc.VectorSubcoreMesh(core_axis_name="c", subcore_axis_name="s")

    @functools.partial(
        pl.kernel,
        mesh=mesh,
        compiler_params=pltpu.CompilerParams(use_tc_tiling_on_sc=False),
        out_type=jax.ShapeDtypeStruct((B, L, E), jnp.float32),
        scratch_types=[
            pltpu.VMEM((rows_per_w, L), jnp.int32),
            pltpu.VMEM_SHARED((L, E), jnp.float32),
            [pltpu.VMEM((L, E), jnp.float32)] * _NBUF,
            [pltpu.SemaphoreType.DMA] * _NBUF,
            [pltpu.SemaphoreType.DMA] * _NBUF,
        ],
    )
    def emb_kernel(inputs_hbm, table_hbm, pos_hbm, out_hbm,
                   idx_v, pos_sh, bufs, gsems, wsems):
        sid = lax.axis_index("s")
        wid = sid * 2 + lax.axis_index("c")
        base = wid * rows_per_w

        pltpu.sync_copy(inputs_hbm.at[pl.ds(base, rows_per_w)], idx_v)

        @pl.when(sid == 0)
        def _stage_pos():
            pltpu.sync_copy(pos_hbm, pos_sh)

        plsc.subcore_barrier()

        def gather_copies(r, k):
            cps = [pltpu.make_async_copy(
                table_hbm.at[idx_v.at[r, pl.ds(0, c0)]],
                bufs[k].at[pl.ds(0, c0)], gsems[k])]
            if c1 > 0:
                cps.append(pltpu.make_async_copy(
                    table_hbm.at[idx_v.at[r, pl.ds(c0, c1)]],
                    bufs[k].at[pl.ds(c0, c1)], gsems[k]))
            return cps

        def wb_copy(r, k):
            return pltpu.make_async_copy(bufs[k], out_hbm.at[base + r],
                                         wsems[k])

        def body(ib, carry):
            for k in range(_NBUF):
                r = ib * _NBUF + k

                @pl.when(ib > 0)
                def _drain_wb():
                    wb_copy(r - _NBUF, k).wait()

                pltpu.sync_copy(pos_sh, bufs[k])
                for cp in gather_copies(r, k):
                    cp.start(add=True)
            for k in range(_NBUF):
                r = ib * _NBUF + k
                for cp in gather_copies(r, k):
                    cp.wait()
                wb_copy(r, k).start()
            return carry

        lax.fori_loop(0, blocks, body, 0)
        for k in range(_NBUF):
            wb_copy(rows_per_w - _NBUF + k, k).wait()

    return emb_kernel(inputs, token_table, pos_table)
